# baseline (speedup 1.0000x reference)
import functools

import jax
import jax.numpy as jnp
from jax import lax
from jax.experimental import pallas as pl
from jax.experimental.pallas import tpu as pltpu
from jax.experimental.pallas import tpu_sc as plsc

_N = 4096
_E = 262144
_C = 4096
_NEG = float("-inf")

_DN = lax.GatherDimensionNumbers(
    offset_dims=(), collapsed_slice_dims=(0,), start_index_map=(0,)
)


def _pad16(n):
    return (n + 15) // 16 * 16


def _bcast(v, e):
    return lax.gather(
        v,
        jnp.full((16, 1), e, jnp.int32),
        _DN,
        (1,),
        mode=lax.GatherScatterMode.PROMISE_IN_BOUNDS,
    )


def _scalar32(va, vb, w):
    val = jnp.int32(0)
    for k in range(16):
        val = jnp.where(w == k, va[k], val)
        val = jnp.where(w == k + 16, vb[k], val)
    return val


@functools.lru_cache(None)
def _chunk_for(S):
    return 4096 if S == 1 else 8192


def _segmax_sc(coutp):
    S = coutp // 16
    P = 32 // S
    R = _N // P
    C = _chunk_for(S)
    mesh = plsc.VectorSubcoreMesh(core_axis_name="c", subcore_axis_name="s")

    R2 = R // 2
    GW = 128

    def body(a_hbm, edges_hbm, meta_hbm, out_hbm, a_v, acc_v, eba_v, ebb_v, meta_v):
        c = lax.axis_index("c")
        s = lax.axis_index("s")
        w = s * 2 + c
        part = w // S
        sl = w % S
        row_lo = pl.multiple_of(part * R, R)
        pltpu.sync_copy(meta_hbm, meta_v)
        pltpu.sync_copy(a_hbm.at[sl], a_v)
        saa = _scalar32(meta_v[0:16], meta_v[16:32], w)
        nca = _scalar32(meta_v[32:48], meta_v[48:64], w)
        sab = _scalar32(meta_v[64:80], meta_v[80:96], w)
        ncb = _scalar32(meta_v[96:112], meta_v[112:128], w)
        gmax = jnp.maximum(nca, ncb)

        def ini(r, carry):
            acc_v[pl.ds(r * 16, 16)] = jnp.full((16,), _NEG, jnp.float32)
            return carry

        lax.fori_loop(0, R + 2 * GW // 16, ini, 0)
        iota = lax.broadcasted_iota(jnp.int32, (16,), 0)
        row_lo16 = jnp.full((16,), 16, jnp.int32) * row_lo
        row_mid16 = row_lo16 + (R2 * 16)
        boff = jnp.full((16,), R2 * 16 + GW, jnp.int32)

        def edge(wv, m, dprev, base16, shift):
            a = plsc.load_gather(a_v, [(wv & 0x1FFF0) | iota])
            dv16 = lax.shift_right_logical(wv, 13) & 0x1FFF0
            mm = jnp.maximum(a, jnp.where(dv16 == dprev, m, _NEG))
            diff = plsc.bitcast(dv16 - base16, jnp.uint32)
            cid = plsc.bitcast(jnp.minimum(diff, jnp.uint32(R2 * 16)), jnp.int32)
            cidx = (cid + shift) | iota if shift is not None else cid | iota
            plsc.store_scatter(acc_v, [cidx], mm)
            return mm, dv16

        def chunk(gi, carry):
            ma, da, mb, db = carry
            offa = pl.multiple_of(jnp.minimum(saa + gi * C, _E), 16)
            pltpu.sync_copy(edges_hbm.at[pl.ds(offa, C)], eba_v)
            offb = pl.multiple_of(jnp.minimum(sab + gi * C, _E), 16)
            pltpu.sync_copy(edges_hbm.at[pl.ds(offb, C)], ebb_v)

            def group(q, carry):
                ma, da, mb, db = carry
                eva = eba_v[pl.ds(q * 16, 16)]
                evb = ebb_v[pl.ds(q * 16, 16)]
                for e in range(16):
                    ma, da = edge(_bcast(eva, e), ma, da, row_lo16, None)
                    mb, db = edge(_bcast(evb, e), mb, db, row_mid16, boff)
                return ma, da, mb, db

            return lax.fori_loop(0, C // 16, group, (ma, da, mb, db))

        negv = jnp.full((16,), _NEG, jnp.float32)
        d0 = jnp.full((16,), -1, jnp.int32)
        lax.fori_loop(0, gmax, chunk, (negv, d0, negv, d0))
        pltpu.sync_copy(
            acc_v.at[pl.ds(0, R2 * 16)],
            out_hbm.at[sl, pl.ds(pl.multiple_of(row_lo * 16, 1024), R2 * 16)],
        )
        pltpu.sync_copy(
            acc_v.at[pl.ds(R2 * 16 + GW, R2 * 16)],
            out_hbm.at[sl, pl.ds(pl.multiple_of((row_lo + R2) * 16, 1024), R2 * 16)],
        )

    return pl.kernel(
        body,
        out_type=jax.ShapeDtypeStruct((S, _N * 16), jnp.float32),
        mesh=mesh,
        compiler_params=pltpu.CompilerParams(needs_layout_passes=False),
        scratch_types=[
            pltpu.VMEM((_N * 16,), jnp.float32),
            pltpu.VMEM((R * 16 + 2 * GW,), jnp.float32),
            pltpu.VMEM((C,), jnp.int32),
            pltpu.VMEM((C,), jnp.int32),
            pltpu.VMEM((128,), jnp.int32),
        ],
    )


def _meta_for(sorted24, S):
    P = 32 // S
    R = _N // P
    R2 = R // 2
    C = _chunk_for(S)
    keys = (jnp.arange(2 * P + 1, dtype=jnp.int32) * R2) << 12
    bnd = jnp.searchsorted(sorted24, keys, side="left").astype(jnp.int32)
    w = jnp.arange(32, dtype=jnp.int32)
    part = w // S
    saa = bnd[2 * part] & ~15
    nca = jnp.maximum(1, (bnd[2 * part + 1] - saa + C - 1) // C)
    sab = bnd[2 * part + 1] & ~15
    ncb = jnp.maximum(1, (bnd[2 * part + 2] - sab + C - 1) // C)
    return jnp.concatenate([saa, nca, sab, ncb])


def _ab_body(g_ref, bp_ref, wt_ref, wc_ref, bs_ref, a_ref, b_ref):
    g = g_ref[...]
    h = jnp.where(jnp.isneginf(g), 0.0, g + bp_ref[...])
    a_ref[...] = jnp.dot(
        h,
        wt_ref[...],
        preferred_element_type=jnp.float32,
        precision=lax.Precision.HIGHEST,
    )
    b_ref[...] = (
        jnp.dot(
            h,
            wc_ref[...],
            preferred_element_type=jnp.float32,
            precision=lax.Precision.HIGHEST,
        )
        + bs_ref[...]
    )


def _ab(agg, b, wt, wc, bs):
    cinp, coutp = wt.shape
    rb = 1024
    return pl.pallas_call(
        _ab_body,
        grid=(_N // rb,),
        in_specs=[
            pl.BlockSpec((rb, cinp), lambda i: (i, 0)),
            pl.BlockSpec((rb, cinp), lambda i: (i, 0)),
            pl.BlockSpec((cinp, coutp), lambda i: (0, 0)),
            pl.BlockSpec((cinp, coutp), lambda i: (0, 0)),
            pl.BlockSpec((1, coutp), lambda i: (0, 0)),
        ],
        out_specs=[
            pl.BlockSpec((rb, coutp), lambda i: (i, 0)),
            pl.BlockSpec((rb, coutp), lambda i: (i, 0)),
        ],
        out_shape=[
            jax.ShapeDtypeStruct((_N, coutp), jnp.float32),
            jax.ShapeDtypeStruct((_N, coutp), jnp.float32),
        ],
    )(agg, b, wt, wc, bs)


def _comb_body(g_ref, bp_ref, h_ref):
    g = g_ref[...]
    h_ref[...] = jnp.where(jnp.isneginf(g), 0.0, g + bp_ref[...])


def _comb(agg, b):
    n, cp = agg.shape
    return pl.pallas_call(
        _comb_body,
        grid=(4,),
        in_specs=[
            pl.BlockSpec((n // 4, cp), lambda i: (i, 0)),
            pl.BlockSpec((n // 4, cp), lambda i: (i, 0)),
        ],
        out_specs=pl.BlockSpec((n // 4, cp), lambda i: (i, 0)),
        out_shape=jax.ShapeDtypeStruct((n, cp), jnp.float32),
    )(agg, b)


def _gram_body(e_ref, w_ref, o_ref):
    o_ref[...] = jnp.dot(
        e_ref[...],
        w_ref[...],
        preferred_element_type=jnp.float32,
        precision=lax.Precision.HIGHEST,
    )


def _gram(ecat, wint):
    rb, cb = 512, 1536
    out = pl.pallas_call(
        _gram_body,
        grid=(_N // rb, (3 * _N) // cb),
        in_specs=[
            pl.BlockSpec((rb, 24), lambda i, j: (i, 0)),
            pl.BlockSpec((24, cb), lambda i, j: (0, j)),
        ],
        out_specs=pl.BlockSpec((rb, cb), lambda i, j: (i, j)),
        out_shape=jax.ShapeDtypeStruct((_N, 3 * _N), jnp.float32),
    )(ecat, wint)
    return out.reshape(_N, _N, 3)


def _pad_params(p):
    cin, cout = p["Wt"].shape
    cinp, coutp = _pad16(cin), _pad16(cout)
    wt = jnp.zeros((cinp, coutp), jnp.float32).at[:cin, :cout].set(p["Wt"])
    wc = (
        jnp.zeros((cinp, coutp), jnp.float32)
        .at[:cin, :cout]
        .set(p["Wp"] - p["Wt"])
    )
    bs = (
        jnp.zeros((1, coutp), jnp.float32)
        .at[0, :cout]
        .set(p["bt"] + p["bp"])
    )
    return wt, wc, bs


@functools.lru_cache(None)
def _segmax_cached(coutp):
    return _segmax_sc(coutp)


def kernel(x, params, edge_index):
    src = edge_index[0]
    dst = edge_index[1]
    packed = (dst << 12) | src
    sorted24 = jnp.sort(packed)
    edges3 = jnp.concatenate(
        [
            ((sorted24 & ~jnp.int32(4095)) << 5) | ((sorted24 & 4095) << 4),
            jnp.full((2 * 8192 + 16,), -1, jnp.int32),
        ]
    )
    metas = {s: _meta_for(sorted24, s) for s in (1, 2, 4, 8)}

    def step(state, p):
        agg, b = state
        wt, wc, bs = _pad_params(p)
        a, b2 = _ab(agg, b, wt, wc, bs)
        coutp = wt.shape[1]
        s_cnt = coutp // 16
        a3 = a.reshape(_N, s_cnt, 16).transpose(1, 0, 2).reshape(s_cnt, _N * 16)
        agg3 = _segmax_cached(coutp)(a3, edges3, metas[s_cnt])
        agg2 = agg3.reshape(s_cnt, _N, 16).transpose(1, 0, 2).reshape(_N, coutp)
        return agg2, b2

    state = (x, jnp.zeros((_N, 128), jnp.float32))
    for p in params["shared"]:
        state = step(state, p)
    st = {name: state for name in ("node", "e1", "e2", "e3")}
    for depth in range(4):
        for name in ("node", "e1", "e2", "e3"):
            if depth < len(params[name]):
                st[name] = step(st[name], params[name][depth])
    finals = {name: _comb(*st[name]) for name in ("node", "e1", "e2", "e3")}
    n_out = finals["node"][:, :7]
    e1, e2, e3 = (finals[k][:, :8] for k in ("e1", "e2", "e3"))
    ecat = jnp.concatenate([e1, e2, e3], axis=1)
    wint = jnp.zeros((3, _N, 3, 8), jnp.float32)
    wint = wint.at[0, :, 0, :].set(e1)
    wint = wint.at[1, :, 1, :].set(e2)
    wint = wint.at[2, :, 2, :].set(e3)
    wint = wint.transpose(0, 3, 1, 2).reshape(24, 3 * _N)
    m = _gram(ecat, wint)
    return (n_out, m)


# speedup vs baseline: 1.1844x; 1.1844x over previous
import functools

import jax
import jax.numpy as jnp
from jax import lax
from jax.experimental import pallas as pl
from jax.experimental.pallas import tpu as pltpu
from jax.experimental.pallas import tpu_sc as plsc

_N = 4096
_E = 262144
_C = 4096
_NEG = float("-inf")

_DN = lax.GatherDimensionNumbers(
    offset_dims=(), collapsed_slice_dims=(0,), start_index_map=(0,)
)


def _pad16(n):
    return (n + 15) // 16 * 16


def _bcast(v, e):
    return lax.gather(
        v,
        jnp.full((16, 1), e, jnp.int32),
        _DN,
        (1,),
        mode=lax.GatherScatterMode.PROMISE_IN_BOUNDS,
    )


def _scalar32(va, vb, w):
    val = jnp.int32(0)
    for k in range(16):
        val = jnp.where(w == k, va[k], val)
        val = jnp.where(w == k + 16, vb[k], val)
    return val


@functools.lru_cache(None)
def _segmax_sc(coutp):
    S = coutp // 16
    P = 32 // S
    R = _N // P
    mesh = plsc.VectorSubcoreMesh(core_axis_name="c", subcore_axis_name="s")

    def body(a_hbm, edges_hbm, meta_hbm, out_hbm, a_v, acc_v, ebuf_v, meta_v):
        c = lax.axis_index("c")
        s = lax.axis_index("s")
        w = s * 2 + c
        part = w // S
        sl = w % S
        row_lo = pl.multiple_of(part * R, R)
        pltpu.sync_copy(meta_hbm, meta_v)
        pltpu.sync_copy(a_hbm.at[sl], a_v)
        sa = _scalar32(meta_v[0:16], meta_v[16:32], w)
        nch = _scalar32(meta_v[32:48], meta_v[48:64], w)

        def ini(r, carry):
            acc_v[pl.ds(r * 16, 16)] = jnp.full((16,), _NEG, jnp.float32)
            return carry

        lax.fori_loop(0, R, ini, 0)
        iota = lax.broadcasted_iota(jnp.int32, (16,), 0)

        def chunk(g, carry):
            m0, d0 = carry
            off = pl.multiple_of(sa + g * _C, 16)
            pltpu.sync_copy(edges_hbm.at[pl.ds(off, _C)], ebuf_v)

            def group(q, carry):
                m, dprev = carry
                ev = ebuf_v[pl.ds(q * 16, 16)]
                for e in range(16):
                    wv = _bcast(ev, e)
                    dv = wv >> 12
                    sv = wv & 4095
                    a = plsc.load_gather(a_v, [(sv << 4) + iota])
                    mm = jnp.maximum(a, jnp.where(dv == dprev, m, _NEG))
                    ridx = dv - row_lo
                    ok = (ridx >= 0) & (ridx < R)
                    cidx = (jnp.clip(ridx, 0, R - 1) << 4) + iota
                    plsc.store_scatter(acc_v, [cidx], mm, mask=ok)
                    m, dprev = mm, dv
                return m, dprev

            return lax.fori_loop(0, _C // 16, group, (m0, d0))

        lax.fori_loop(
            0,
            nch,
            chunk,
            (jnp.full((16,), _NEG, jnp.float32), jnp.full((16,), -1, jnp.int32)),
        )
        pltpu.sync_copy(
            acc_v, out_hbm.at[sl, pl.ds(pl.multiple_of(row_lo * 16, 2048), R * 16)]
        )

    return pl.kernel(
        body,
        out_type=jax.ShapeDtypeStruct((S, _N * 16), jnp.float32),
        mesh=mesh,
        compiler_params=pltpu.CompilerParams(needs_layout_passes=False),
        scratch_types=[
            pltpu.VMEM((_N * 16,), jnp.float32),
            pltpu.VMEM((R * 16,), jnp.float32),
            pltpu.VMEM((_C,), jnp.int32),
            pltpu.VMEM((64,), jnp.int32),
        ],
    )


def _meta_for(sorted_packed, S):
    P = 32 // S
    R = _N // P
    keys = (jnp.arange(P + 1, dtype=jnp.int32) * R) << 12
    bnd = jnp.searchsorted(sorted_packed, keys, side="left").astype(jnp.int32)
    w = jnp.arange(32, dtype=jnp.int32)
    part = w // S
    start = bnd[part]
    end = bnd[part + 1]
    sa = start & ~15
    nch = (end - sa + _C - 1) // _C
    return jnp.concatenate([sa, nch])


def _ab_body(g_ref, bp_ref, wt_ref, wc_ref, bs_ref, a_ref, b_ref):
    g = g_ref[...]
    h = jnp.where(jnp.isneginf(g), 0.0, g + bp_ref[...])
    a_ref[...] = jnp.dot(
        h,
        wt_ref[...],
        preferred_element_type=jnp.float32,
        precision=lax.Precision.HIGHEST,
    )
    b_ref[...] = (
        jnp.dot(
            h,
            wc_ref[...],
            preferred_element_type=jnp.float32,
            precision=lax.Precision.HIGHEST,
        )
        + bs_ref[...]
    )


def _ab(agg, b, wt, wc, bs):
    cinp, coutp = wt.shape
    rb = 512
    return pl.pallas_call(
        _ab_body,
        grid=(_N // rb,),
        in_specs=[
            pl.BlockSpec((rb, cinp), lambda i: (i, 0)),
            pl.BlockSpec((rb, cinp), lambda i: (i, 0)),
            pl.BlockSpec((cinp, coutp), lambda i: (0, 0)),
            pl.BlockSpec((cinp, coutp), lambda i: (0, 0)),
            pl.BlockSpec((1, coutp), lambda i: (0, 0)),
        ],
        out_specs=[
            pl.BlockSpec((rb, coutp), lambda i: (i, 0)),
            pl.BlockSpec((rb, coutp), lambda i: (i, 0)),
        ],
        out_shape=[
            jax.ShapeDtypeStruct((_N, coutp), jnp.float32),
            jax.ShapeDtypeStruct((_N, coutp), jnp.float32),
        ],
    )(agg, b, wt, wc, bs)


def _comb_body(g_ref, bp_ref, h_ref):
    g = g_ref[...]
    h_ref[...] = jnp.where(jnp.isneginf(g), 0.0, g + bp_ref[...])


def _comb(agg, b):
    n, cp = agg.shape
    return pl.pallas_call(
        _comb_body,
        grid=(4,),
        in_specs=[
            pl.BlockSpec((n // 4, cp), lambda i: (i, 0)),
            pl.BlockSpec((n // 4, cp), lambda i: (i, 0)),
        ],
        out_specs=pl.BlockSpec((n // 4, cp), lambda i: (i, 0)),
        out_shape=jax.ShapeDtypeStruct((n, cp), jnp.float32),
    )(agg, b)


def _gram_body(e_ref, w_ref, o_ref):
    o_ref[...] = jnp.dot(
        e_ref[...],
        w_ref[...],
        preferred_element_type=jnp.float32,
        precision=lax.Precision.HIGHEST,
    )


def _gram(ecat, wint):
    rb, cb = 512, 1536
    out = pl.pallas_call(
        _gram_body,
        grid=(_N // rb, (3 * _N) // cb),
        in_specs=[
            pl.BlockSpec((rb, 24), lambda i, j: (i, 0)),
            pl.BlockSpec((24, cb), lambda i, j: (0, j)),
        ],
        out_specs=pl.BlockSpec((rb, cb), lambda i, j: (i, j)),
        out_shape=jax.ShapeDtypeStruct((_N, 3 * _N), jnp.float32),
    )(ecat, wint)
    return out.reshape(_N, _N, 3)


def _pad_params(p):
    cin, cout = p["Wt"].shape
    cinp, coutp = _pad16(cin), _pad16(cout)
    wt = jnp.zeros((cinp, coutp), jnp.float32).at[:cin, :cout].set(p["Wt"])
    wc = (
        jnp.zeros((cinp, coutp), jnp.float32)
        .at[:cin, :cout]
        .set(p["Wp"] - p["Wt"])
    )
    bs = (
        jnp.zeros((1, coutp), jnp.float32)
        .at[0, :cout]
        .set(p["bt"] + p["bp"])
    )
    return wt, wc, bs


def kernel(x, params, edge_index):
    src = edge_index[0]
    dst = edge_index[1]
    packed = (dst << 12) | src
    sorted_packed = jnp.sort(packed)
    edges = jnp.concatenate([sorted_packed, jnp.full((_C,), -1, jnp.int32)])
    metas = {s: _meta_for(sorted_packed, s) for s in (1, 2, 4, 8)}

    def step(state, p):
        agg, b = state
        wt, wc, bs = _pad_params(p)
        a, b2 = _ab(agg, b, wt, wc, bs)
        coutp = wt.shape[1]
        s_cnt = coutp // 16
        a3 = a.reshape(_N, s_cnt, 16).transpose(1, 0, 2).reshape(s_cnt, _N * 16)
        agg3 = _segmax_sc(coutp)(a3, edges, metas[s_cnt])
        agg2 = agg3.reshape(s_cnt, _N, 16).transpose(1, 0, 2).reshape(_N, coutp)
        return agg2, b2

    state = (x, jnp.zeros((_N, 128), jnp.float32))
    for p in params["shared"]:
        state = step(state, p)
    finals = {}
    for name in ("node", "e1", "e2", "e3"):
        st = state
        for p in params[name]:
            st = step(st, p)
        finals[name] = _comb(st[0], st[1])
    n_out = finals["node"][:, :7]
    e1, e2, e3 = (finals[k][:, :8] for k in ("e1", "e2", "e3"))
    ecat = jnp.concatenate([e1, e2, e3], axis=1)
    wint = jnp.zeros((3, _N, 3, 8), jnp.float32)
    wint = wint.at[0, :, 0, :].set(e1)
    wint = wint.at[1, :, 1, :].set(e2)
    wint = wint.at[2, :, 2, :].set(e3)
    wint = wint.transpose(0, 3, 1, 2).reshape(24, 3 * _N)
    m = _gram(ecat, wint)
    return (n_out, m)
